# baseline (speedup 1.0000x reference)
import functools as _ft

import jax
import jax.numpy as jnp
from jax import lax
from jax.experimental import pallas as pl
from jax.experimental.pallas import tpu as pltpu
from jax.experimental.pallas import tpu_sc as plsc

N = 10000
NP = 10240
DUMMY = N
E = 320000
NC, NS = 2, 16
NW = NC * NS
CH = 64
K = 160
IB = 16
NB = K // IB
EPAD = NW * CH * K
NIR = EPAD // 128
RB = 512
RPT = NP // NS
NPD = NP // 16
F32 = jnp.float32


def _pre_body(x_ref, w1_ref, as_ref, ad_ref, h_ref, s_ref, d_ref):
    h = jnp.dot(x_ref[...], w1_ref[...], preferred_element_type=F32)
    h_ref[...] = h
    s_ref[...] = jnp.dot(h, as_ref[...], preferred_element_type=F32)
    d_ref[...] = jnp.dot(h, ad_ref[...], preferred_element_type=F32)


def _tc_pre(xp, W1, A_s1, A_d1):
    return pl.pallas_call(
        _pre_body,
        grid=(NP // RB,),
        in_specs=[
            pl.BlockSpec((RB, 128), lambda i: (i, 0)),
            pl.BlockSpec((128, 128), lambda i: (0, 0)),
            pl.BlockSpec((128, 128), lambda i: (0, 0)),
            pl.BlockSpec((128, 128), lambda i: (0, 0)),
        ],
        out_specs=[
            pl.BlockSpec((RB, 128), lambda i: (i, 0)),
            pl.BlockSpec((RB, 128), lambda i: (i, 0)),
            pl.BlockSpec((RB, 128), lambda i: (i, 0)),
        ],
        out_shape=[
            jax.ShapeDtypeStruct((NP, 128), F32),
            jax.ShapeDtypeStruct((NP, 128), F32),
            jax.ShapeDtypeStruct((NP, 128), F32),
        ],
    )(xp, W1, A_s1, A_d1)


def _mid_body(acc_ref, den_ref, b1_ref, w2e_ref, w2d_ref, e8_ref, p_ref,
              c40_ref, src_ref, dst_ref):
    num = acc_ref[0] + acc_ref[1]
    dp = den_ref[0] + den_ref[1]
    parts = [
        jnp.dot(dp[:, 8 * p:8 * p + 8], e8_ref[...],
                preferred_element_type=F32)
        for p in range(16)
    ]
    cat = jnp.concatenate(parts, axis=0)
    dex = jnp.dot(p_ref[...], cat, preferred_element_type=F32)
    v = num / (dex + 1e-16) + b1_ref[...]
    hmid = jnp.where(v > 0, v, jnp.exp(v) - 1.0)
    src_ref[...] = jnp.dot(hmid, w2e_ref[...],
                           preferred_element_type=F32) + c40_ref[...]
    dst_ref[...] = jnp.dot(hmid, w2d_ref[...], preferred_element_type=F32)


def _tc_mid(acc1, den1, b1r, W2E, W2D, E8b, P2, c40):
    return pl.pallas_call(
        _mid_body,
        grid=(NP // RB,),
        in_specs=[
            pl.BlockSpec((NC, RB, 128), lambda i: (0, i, 0)),
            pl.BlockSpec((NC, RB // 16, 128), lambda i: (0, i, 0)),
            pl.BlockSpec((1, 128), lambda i: (0, 0)),
            pl.BlockSpec((128, 128), lambda i: (0, 0)),
            pl.BlockSpec((128, 128), lambda i: (0, 0)),
            pl.BlockSpec((8, 128), lambda i: (0, 0)),
            pl.BlockSpec((RB, RB), lambda i: (0, 0)),
            pl.BlockSpec((1, 128), lambda i: (0, 0)),
        ],
        out_specs=[
            pl.BlockSpec((RB, 128), lambda i: (i, 0)),
            pl.BlockSpec((RB, 128), lambda i: (i, 0)),
        ],
        out_shape=[
            jax.ShapeDtypeStruct((NP, 128), F32),
            jax.ShapeDtypeStruct((NP, 128), F32),
        ],
    )(acc1, den1, b1r, W2E, W2D, E8b, P2, c40)


def _out_body(acc_ref, b2_ref, o_ref):
    num = acc_ref[0] + acc_ref[1]
    o_ref[...] = num[:, :40] / (num[:, 40:41] + 1e-16) + b2_ref[...]


def _tc_out(acc2, b2r):
    return pl.pallas_call(
        _out_body,
        grid=(NP // RB,),
        in_specs=[
            pl.BlockSpec((NC, RB, 128), lambda i: (0, i, 0)),
            pl.BlockSpec((1, 40), lambda i: (0, 0)),
        ],
        out_specs=pl.BlockSpec((RB, 40), lambda i: (i, 0)),
        out_shape=jax.ShapeDtypeStruct((NP, 40), F32),
    )(acc2, b2r)


def _edge_pass(sep_logits, s_off, multi_head, use_den, CHL=CH):
    D = 128
    NV = D // 16
    mesh = plsc.VectorSubcoreMesh(
        core_axis_name="c", subcore_axis_name="s",
        num_cores=NC, num_subcores=NS)
    outs = [jax.ShapeDtypeStruct((NC * NP, D), F32)]
    if use_den:
        outs.append(jax.ShapeDtypeStruct((NC * NPD, D), F32))
    scratch = [
        pltpu.VMEM_SHARED((NP, D), F32),
        pltpu.VMEM((8, 128), jnp.int32),
        pltpu.VMEM((8, 128), jnp.int32),
        pltpu.VMEM((1, CHL), jnp.int32),
        pltpu.VMEM((1, CHL), jnp.int32),
        pltpu.VMEM((1, 128), jnp.int32),
        pltpu.VMEM((CHL, D), F32),
        pltpu.VMEM((CHL, D), F32),
        pltpu.SemaphoreType.DMA,
        pltpu.SemaphoreType.DMA,
        pltpu.SemaphoreType.DMA,
        pltpu.SemaphoreType.DMA,
        pltpu.SemaphoreType.DMA,
    ]
    if sep_logits:
        scratch.append(pltpu.VMEM((CHL, D), F32))
    if use_den:
        scratch.append(pltpu.VMEM_SHARED((NPD, D), F32))
        scratch.append(pltpu.VMEM((1, CHL), jnp.int32))

    def body(srcI, dstI, stab, atab, dtab, *refs):
        nout = 2 if use_den else 1
        out_acc = refs[0]
        out_den = refs[1] if use_den else None
        refs = refs[nout:]
        (acc_sh, sidx_blk, didx_blk, sidx_c, didx_c, ridx, h_v, ad_v,
         sem_h, sem_as, sem_ad, sem_acc, sem_den) = refs[:13]
        refs = refs[13:]
        if sep_logits:
            as_v = refs[0]
            refs = refs[1:]
        if use_den:
            denp_sh, didx16_c = refs
        c = lax.axis_index("c")
        s = lax.axis_index("s")
        w = c * NS + s

        def zrow(e, carry):
            for j in range(NV):
                h_v[e, pl.ds(16 * j, 16)] = jnp.zeros((16,), F32)
            return carry
        lax.fori_loop(0, CHL, zrow, None)
        for i in range(RPT // CHL):
            pltpu.sync_copy(h_v, acc_sh.at[pl.ds(s * RPT + i * CHL, CHL)])
        if use_den:
            @pl.when(s < NPD // CHL)
            def _():
                pltpu.sync_copy(h_v, denp_sh.at[pl.ds(s * CHL, CHL)])
        plsc.subcore_barrier()

        nb_w = jnp.where(c == 0, 8, 12)
        row0 = jnp.where(c == 0, s * 64, 1024 + s * 96)

        def block(b, carry):
            ridx[0, pl.ds(0, 16)] = lax.iota(jnp.int32, 16) + (row0 + 8 * b)
            pltpu.sync_copy(srcI.at[ridx.at[0, pl.ds(0, 8)]], sidx_blk)
            pltpu.sync_copy(dstI.at[ridx.at[0, pl.ds(0, 8)]], didx_blk)

            def chunk(ci, carry1):
                @pl.when((b * IB + ci) > 0)
                def _():
                    pltpu.make_async_copy(
                        h_v, acc_sh.at[didx_c.at[0]], sem_acc).wait()
                    if use_den:
                        pltpu.make_async_copy(
                            ad_v, denp_sh.at[didx16_c.at[0]], sem_den).wait()
                flat = ci * CHL
                for t in range(CHL // 16):
                    fl = flat + 16 * t
                    r = lax.shift_right_logical(fl, 7)
                    off = fl & 127
                    sv = sidx_blk[r, pl.ds(off, 16)]
                    dv = didx_blk[r, pl.ds(off, 16)]
                    sidx_c[0, pl.ds(16 * t, 16)] = sv
                    didx_c[0, pl.ds(16 * t, 16)] = dv
                    if use_den:
                        didx16_c[0, pl.ds(16 * t, 16)] = \
                            lax.shift_right_logical(dv, 4)
                pltpu.async_copy(stab.at[sidx_c.at[0]], h_v, sem_h)
                if sep_logits:
                    pltpu.async_copy(atab.at[sidx_c.at[0]], as_v, sem_as)
                pltpu.async_copy(dtab.at[didx_c.at[0]], ad_v, sem_ad)
                pltpu.make_async_copy(stab.at[sidx_c.at[0]], h_v,
                                      sem_h).wait()
                if sep_logits:
                    pltpu.make_async_copy(atab.at[sidx_c.at[0]], as_v,
                                          sem_as).wait()
                pltpu.make_async_copy(dtab.at[didx_c.at[0]], ad_v,
                                      sem_ad).wait()

                def group(g, carry2):
                    dvec = didx_c[0, pl.ds(16 * g, 16)]
                    for v in range(16):
                        e = 16 * g + v
                        if sep_logits:
                            t = as_v[e, pl.ds(0, 16)] + ad_v[e, pl.ds(0, 16)]
                        else:
                            t = h_v[e, pl.ds(s_off, 16)] + \
                                ad_v[e, pl.ds(0, 16)]
                        wv = jnp.exp(jnp.maximum(t, 0.2 * t))
                        if use_den:
                            io16 = lax.iota(jnp.int32, 16)
                            mask8 = jnp.minimum(
                                jnp.maximum(8 - io16, 0), 1).astype(F32)
                            wc = wv * mask8
                            rot8 = (io16 + 8) & 15
                            sh = jnp.take_along_axis(
                                wc, rot8, axis=0, mode="promise_in_bounds")
                            dj = dvec[v]
                            base = (dj & 14) * 8
                            parf = jnp.full(
                                (16,), dj & 1, jnp.int32).astype(F32)
                            val = wc + (sh - wc) * parf
                            ad_v[e, pl.ds(0, 16)] = jnp.zeros((16,), F32)
                            ad_v[e, pl.ds(base, 16)] = val
                        for j in range(NV):
                            col = j if multi_head else 0
                            bb = jnp.take_along_axis(
                                wv, jnp.full((16,), col, jnp.int32), axis=0,
                                mode="promise_in_bounds")
                            h_v[e, pl.ds(16 * j, 16)] = \
                                h_v[e, pl.ds(16 * j, 16)] * bb
                    return carry2
                lax.fori_loop(0, CHL // 16, group, None)

                pltpu.async_copy(h_v, acc_sh.at[didx_c.at[0]], sem_acc,
                                 add=True)
                if use_den:
                    pltpu.async_copy(ad_v, denp_sh.at[didx16_c.at[0]],
                                     sem_den, add=True)
                return carry1
            lax.fori_loop(0, 1024 // CHL, chunk, None)
            return carry
        lax.fori_loop(0, nb_w, block, None)
        pltpu.make_async_copy(h_v, acc_sh.at[didx_c.at[0]], sem_acc).wait()
        if use_den:
            pltpu.make_async_copy(ad_v, denp_sh.at[didx16_c.at[0]],
                                  sem_den).wait()
        plsc.subcore_barrier()

        for i in range(RPT // CHL):
            r0 = s * RPT + i * CHL
            for t in range(CHL // 16):
                ridx[0, pl.ds(16 * t, 16)] = \
                    lax.iota(jnp.int32, 16) + (c * NP + r0 + 16 * t)
            pltpu.sync_copy(acc_sh.at[pl.ds(r0, CHL)], h_v)
            pltpu.sync_copy(h_v, out_acc.at[ridx.at[0, pl.ds(0, CHL)]])
        if use_den:
            @pl.when(s < NPD // CHL)
            def _():
                r0 = s * CHL
                for t in range(CHL // 16):
                    ridx[0, pl.ds(16 * t, 16)] = \
                        lax.iota(jnp.int32, 16) + (c * NPD + r0 + 16 * t)
                pltpu.sync_copy(denp_sh.at[pl.ds(r0, CHL)], h_v)
                pltpu.sync_copy(h_v, out_den.at[ridx.at[0, pl.ds(0, CHL)]])

    return pl.kernel(body, out_type=tuple(outs) if use_den else outs[0],
                     mesh=mesh, scratch_types=scratch)


@_ft.lru_cache(maxsize=None)
def _edge_pass_cached(sep_logits, s_off, multi_head, use_den, CHL=CH):
    return _edge_pass(sep_logits, s_off, multi_head, use_den, CHL)


def kernel(x, edge_index, W1, att_src1, att_dst1, b1, W2, att_src2,
           att_dst2, b2):
    xp = jnp.zeros((NP, 128), F32).at[:N].set(x)
    pad = jnp.full((EPAD - E,), DUMMY, jnp.int32)
    srcI = jnp.concatenate([edge_index[0], pad]).reshape(NIR, 128)
    dstI = jnp.concatenate([edge_index[1], pad]).reshape(NIR, 128)

    eye8 = jnp.eye(8, dtype=F32)
    A_s1 = jnp.zeros((128, 128), F32).at[:, :8].set(
        (att_src1[:, :, None] * eye8[:, None, :]).reshape(128, 8))
    A_d1 = jnp.zeros((128, 128), F32).at[:, :8].set(
        (att_dst1[:, :, None] * eye8[:, None, :]).reshape(128, 8))
    E8b = (jnp.arange(128)[None, :] // 16 ==
           jnp.arange(8)[:, None]).astype(F32)
    r = jnp.arange(RB)
    P2 = (jnp.arange(RB)[None, :] ==
          ((r % 16) * (RB // 16) + r // 16)[:, None]).astype(F32)
    W2E = jnp.zeros((128, 128), F32).at[:, :40].set(W2)
    W2E = W2E.at[:, 48].set(W2 @ att_src2[0])
    W2D = jnp.zeros((128, 128), F32).at[:, 0].set(W2 @ att_dst2[0])
    c40 = jnp.zeros((1, 128), F32).at[0, 40].set(1.0)

    htab1, astab1, adtab1 = _tc_pre(xp, W1, A_s1, A_d1)
    acc1, den1 = _edge_pass_cached(True, 0, True, True)(
        srcI, dstI, htab1, astab1, adtab1)
    acc1 = acc1.reshape(NC, NP, 128)
    den1 = den1.reshape(NC, NPD, 128)
    stab2, dtab2 = _tc_mid(acc1, den1, b1.reshape(1, 128), W2E, W2D, E8b, P2,
                           c40)
    acc2 = _edge_pass_cached(False, 48, False, False, 128)(
        srcI, dstI, stab2, stab2, dtab2)
    acc2 = acc2.reshape(NC, NP, 128)
    out = _tc_out(acc2, b2.reshape(1, 40))
    return out[:N]


# speedup vs baseline: 1.0925x; 1.0925x over previous
import functools as _ft

import jax
import jax.numpy as jnp
from jax import lax
from jax.experimental import pallas as pl
from jax.experimental.pallas import tpu as pltpu
from jax.experimental.pallas import tpu_sc as plsc

N = 10000
NP = 10240
DUMMY = N
E = 320000
NC, NS = 2, 16
NW = NC * NS
CH = 64
K = 160
IB = 16
NB = K // IB
EPAD = NW * CH * K
NIR = EPAD // 128
RB = 512
RPT = NP // NS
NPD = NP // 16
F32 = jnp.float32


def _pre_body(x_ref, w1_ref, as_ref, ad_ref, h_ref, s_ref, d_ref):
    h = jnp.dot(x_ref[...], w1_ref[...], preferred_element_type=F32)
    h_ref[...] = h
    s_ref[...] = jnp.dot(h, as_ref[...], preferred_element_type=F32)
    d_ref[...] = jnp.dot(h, ad_ref[...], preferred_element_type=F32)


def _tc_pre(xp, W1, A_s1, A_d1):
    return pl.pallas_call(
        _pre_body,
        grid=(NP // RB,),
        in_specs=[
            pl.BlockSpec((RB, 128), lambda i: (i, 0)),
            pl.BlockSpec((128, 128), lambda i: (0, 0)),
            pl.BlockSpec((128, 128), lambda i: (0, 0)),
            pl.BlockSpec((128, 128), lambda i: (0, 0)),
        ],
        out_specs=[
            pl.BlockSpec((RB, 128), lambda i: (i, 0)),
            pl.BlockSpec((RB, 128), lambda i: (i, 0)),
            pl.BlockSpec((RB, 128), lambda i: (i, 0)),
        ],
        out_shape=[
            jax.ShapeDtypeStruct((NP, 128), F32),
            jax.ShapeDtypeStruct((NP, 128), F32),
            jax.ShapeDtypeStruct((NP, 128), F32),
        ],
    )(xp, W1, A_s1, A_d1)


def _mid_body(acc_ref, den_ref, b1_ref, w2e_ref, w2d_ref, e8_ref, p_ref,
              c40_ref, src_ref, dst_ref):
    num = acc_ref[0] + acc_ref[1]
    dp = den_ref[0] + den_ref[1]
    parts = [
        jnp.dot(dp[:, 8 * p:8 * p + 8], e8_ref[...],
                preferred_element_type=F32)
        for p in range(16)
    ]
    cat = jnp.concatenate(parts, axis=0)
    dex = jnp.dot(p_ref[...], cat, preferred_element_type=F32)
    v = num / (dex + 1e-16) + b1_ref[...]
    hmid = jnp.where(v > 0, v, jnp.exp(v) - 1.0)
    src_ref[...] = jnp.dot(hmid, w2e_ref[...],
                           preferred_element_type=F32) + c40_ref[...]
    dst_ref[...] = jnp.dot(hmid, w2d_ref[...], preferred_element_type=F32)


def _tc_mid(acc1, den1, b1r, W2E, W2D, E8b, P2, c40):
    return pl.pallas_call(
        _mid_body,
        grid=(NP // RB,),
        in_specs=[
            pl.BlockSpec((NC, RB, 128), lambda i: (0, i, 0)),
            pl.BlockSpec((NC, RB // 16, 128), lambda i: (0, i, 0)),
            pl.BlockSpec((1, 128), lambda i: (0, 0)),
            pl.BlockSpec((128, 128), lambda i: (0, 0)),
            pl.BlockSpec((128, 128), lambda i: (0, 0)),
            pl.BlockSpec((8, 128), lambda i: (0, 0)),
            pl.BlockSpec((RB, RB), lambda i: (0, 0)),
            pl.BlockSpec((1, 128), lambda i: (0, 0)),
        ],
        out_specs=[
            pl.BlockSpec((RB, 128), lambda i: (i, 0)),
            pl.BlockSpec((RB, 128), lambda i: (i, 0)),
        ],
        out_shape=[
            jax.ShapeDtypeStruct((NP, 128), F32),
            jax.ShapeDtypeStruct((NP, 128), F32),
        ],
    )(acc1, den1, b1r, W2E, W2D, E8b, P2, c40)


def _out_body(acc_ref, b2_ref, o_ref):
    num = acc_ref[0] + acc_ref[1]
    o_ref[...] = num[:, :40] / (num[:, 40:41] + 1e-16) + b2_ref[...]


def _tc_out(acc2, b2r):
    return pl.pallas_call(
        _out_body,
        grid=(NP // RB,),
        in_specs=[
            pl.BlockSpec((NC, RB, 128), lambda i: (0, i, 0)),
            pl.BlockSpec((1, 40), lambda i: (0, 0)),
        ],
        out_specs=pl.BlockSpec((RB, 40), lambda i: (i, 0)),
        out_shape=jax.ShapeDtypeStruct((NP, 40), F32),
    )(acc2, b2r)


def _edge_pass(sep_logits, s_off, multi_head, use_den, CHL=CH):
    D = 128
    NV = D // 16
    mesh = plsc.VectorSubcoreMesh(
        core_axis_name="c", subcore_axis_name="s",
        num_cores=NC, num_subcores=NS)
    outs = [jax.ShapeDtypeStruct((NC * NP, D), F32)]
    if use_den:
        outs.append(jax.ShapeDtypeStruct((NC * NPD, D), F32))
    scratch = [
        pltpu.VMEM_SHARED((NP, D), F32),
        pltpu.VMEM((8, 128), jnp.int32),
        pltpu.VMEM((8, 128), jnp.int32),
        pltpu.VMEM((1, CHL), jnp.int32),
        pltpu.VMEM((1, CHL), jnp.int32),
        pltpu.VMEM((1, 128), jnp.int32),
        pltpu.VMEM((CHL, D), F32),
        pltpu.VMEM((CHL, D), F32),
        pltpu.SemaphoreType.DMA,
        pltpu.SemaphoreType.DMA,
        pltpu.SemaphoreType.DMA,
        pltpu.SemaphoreType.DMA,
        pltpu.SemaphoreType.DMA,
    ]
    if sep_logits:
        scratch.append(pltpu.VMEM((CHL, D), F32))
    if use_den:
        scratch.append(pltpu.VMEM_SHARED((NPD, D), F32))
        scratch.append(pltpu.VMEM((1, CHL), jnp.int32))

    def body(srcI, dstI, stab, atab, dtab, *refs):
        nout = 2 if use_den else 1
        out_acc = refs[0]
        out_den = refs[1] if use_den else None
        refs = refs[nout:]
        (acc_sh, sidx_blk, didx_blk, sidx_c, didx_c, ridx, h_v, ad_v,
         sem_h, sem_as, sem_ad, sem_acc, sem_den) = refs[:13]
        refs = refs[13:]
        if sep_logits:
            as_v = refs[0]
            refs = refs[1:]
        if use_den:
            denp_sh, didx16_c = refs
        c = lax.axis_index("c")
        s = lax.axis_index("s")
        w = c * NS + s

        def zrow(e, carry):
            for j in range(NV):
                h_v[e, pl.ds(16 * j, 16)] = jnp.zeros((16,), F32)
            return carry
        lax.fori_loop(0, CHL, zrow, None)
        for i in range(RPT // CHL):
            pltpu.sync_copy(h_v, acc_sh.at[pl.ds(s * RPT + i * CHL, CHL)])
        if use_den:
            @pl.when(s < NPD // CHL)
            def _():
                pltpu.sync_copy(h_v, denp_sh.at[pl.ds(s * CHL, CHL)])
        plsc.subcore_barrier()

        def block(b, carry):
            ridx[0, pl.ds(0, 16)] = lax.iota(jnp.int32, 16) + (w * 80 + 8 * b)
            pltpu.sync_copy(srcI.at[ridx.at[0, pl.ds(0, 8)]], sidx_blk)
            pltpu.sync_copy(dstI.at[ridx.at[0, pl.ds(0, 8)]], didx_blk)

            def chunk(ci, carry1):
                @pl.when((b * IB + ci) > 0)
                def _():
                    pltpu.make_async_copy(
                        h_v, acc_sh.at[didx_c.at[0]], sem_acc).wait()
                    if use_den:
                        pltpu.make_async_copy(
                            ad_v, denp_sh.at[didx16_c.at[0]], sem_den).wait()
                flat = ci * CHL
                for t in range(CHL // 16):
                    fl = flat + 16 * t
                    r = lax.shift_right_logical(fl, 7)
                    off = fl & 127
                    sv = sidx_blk[r, pl.ds(off, 16)]
                    dv = didx_blk[r, pl.ds(off, 16)]
                    sidx_c[0, pl.ds(16 * t, 16)] = sv
                    didx_c[0, pl.ds(16 * t, 16)] = dv
                    if use_den:
                        didx16_c[0, pl.ds(16 * t, 16)] = \
                            lax.shift_right_logical(dv, 4)
                pltpu.async_copy(stab.at[sidx_c.at[0]], h_v, sem_h)
                if sep_logits:
                    pltpu.async_copy(atab.at[sidx_c.at[0]], as_v, sem_as)
                pltpu.async_copy(dtab.at[didx_c.at[0]], ad_v, sem_ad)
                pltpu.make_async_copy(stab.at[sidx_c.at[0]], h_v,
                                      sem_h).wait()
                if sep_logits:
                    pltpu.make_async_copy(atab.at[sidx_c.at[0]], as_v,
                                          sem_as).wait()
                pltpu.make_async_copy(dtab.at[didx_c.at[0]], ad_v,
                                      sem_ad).wait()

                def group(g, carry2):
                    dvec = didx_c[0, pl.ds(16 * g, 16)]
                    for v in range(16):
                        e = 16 * g + v
                        if sep_logits:
                            t = as_v[e, pl.ds(0, 16)] + ad_v[e, pl.ds(0, 16)]
                        else:
                            t = h_v[e, pl.ds(s_off, 16)] + \
                                ad_v[e, pl.ds(0, 16)]
                        wv = jnp.exp(jnp.maximum(t, 0.2 * t))
                        if use_den:
                            io16 = lax.iota(jnp.int32, 16)
                            mask8 = jnp.minimum(
                                jnp.maximum(8 - io16, 0), 1).astype(F32)
                            wc = wv * mask8
                            rot8 = (io16 + 8) & 15
                            sh = jnp.take_along_axis(
                                wc, rot8, axis=0, mode="promise_in_bounds")
                            dj = dvec[v]
                            base = (dj & 14) * 8
                            parf = jnp.full(
                                (16,), dj & 1, jnp.int32).astype(F32)
                            val = wc + (sh - wc) * parf
                            ad_v[e, pl.ds(0, 16)] = jnp.zeros((16,), F32)
                            ad_v[e, pl.ds(base, 16)] = val
                        for j in range(NV):
                            col = j if multi_head else 0
                            bb = jnp.take_along_axis(
                                wv, jnp.full((16,), col, jnp.int32), axis=0,
                                mode="promise_in_bounds")
                            h_v[e, pl.ds(16 * j, 16)] = \
                                h_v[e, pl.ds(16 * j, 16)] * bb
                    return carry2
                lax.fori_loop(0, CHL // 16, group, None)

                pltpu.async_copy(h_v, acc_sh.at[didx_c.at[0]], sem_acc,
                                 add=True)
                if use_den:
                    pltpu.async_copy(ad_v, denp_sh.at[didx16_c.at[0]],
                                     sem_den, add=True)
                return carry1
            lax.fori_loop(0, 1024 // CHL, chunk, None)
            return carry
        lax.fori_loop(0, NB, block, None)
        pltpu.make_async_copy(h_v, acc_sh.at[didx_c.at[0]], sem_acc).wait()
        if use_den:
            pltpu.make_async_copy(ad_v, denp_sh.at[didx16_c.at[0]],
                                  sem_den).wait()
        plsc.subcore_barrier()

        for i in range(RPT // CHL):
            r0 = s * RPT + i * CHL
            for t in range(CHL // 16):
                ridx[0, pl.ds(16 * t, 16)] = \
                    lax.iota(jnp.int32, 16) + (c * NP + r0 + 16 * t)
            pltpu.sync_copy(acc_sh.at[pl.ds(r0, CHL)], h_v)
            pltpu.sync_copy(h_v, out_acc.at[ridx.at[0, pl.ds(0, CHL)]])
        if use_den:
            @pl.when(s < NPD // CHL)
            def _():
                r0 = s * CHL
                for t in range(CHL // 16):
                    ridx[0, pl.ds(16 * t, 16)] = \
                        lax.iota(jnp.int32, 16) + (c * NPD + r0 + 16 * t)
                pltpu.sync_copy(denp_sh.at[pl.ds(r0, CHL)], h_v)
                pltpu.sync_copy(h_v, out_den.at[ridx.at[0, pl.ds(0, CHL)]])

    return pl.kernel(body, out_type=tuple(outs) if use_den else outs[0],
                     mesh=mesh, scratch_types=scratch)


@_ft.lru_cache(maxsize=None)
def _edge_pass_cached(sep_logits, s_off, multi_head, use_den, CHL=CH):
    return _edge_pass(sep_logits, s_off, multi_head, use_den, CHL)


def kernel(x, edge_index, W1, att_src1, att_dst1, b1, W2, att_src2,
           att_dst2, b2):
    xp = jnp.zeros((NP, 128), F32).at[:N].set(x)
    pad = jnp.full((EPAD - E,), DUMMY, jnp.int32)
    srcI = jnp.concatenate([edge_index[0], pad]).reshape(NIR, 128)
    dstI = jnp.concatenate([edge_index[1], pad]).reshape(NIR, 128)

    eye8 = jnp.eye(8, dtype=F32)
    A_s1 = jnp.zeros((128, 128), F32).at[:, :8].set(
        (att_src1[:, :, None] * eye8[:, None, :]).reshape(128, 8))
    A_d1 = jnp.zeros((128, 128), F32).at[:, :8].set(
        (att_dst1[:, :, None] * eye8[:, None, :]).reshape(128, 8))
    E8b = (jnp.arange(128)[None, :] // 16 ==
           jnp.arange(8)[:, None]).astype(F32)
    r = jnp.arange(RB)
    P2 = (jnp.arange(RB)[None, :] ==
          ((r % 16) * (RB // 16) + r // 16)[:, None]).astype(F32)
    W2E = jnp.zeros((128, 128), F32).at[:, :40].set(W2)
    W2E = W2E.at[:, 48].set(W2 @ att_src2[0])
    W2D = jnp.zeros((128, 128), F32).at[:, 0].set(W2 @ att_dst2[0])
    c40 = jnp.zeros((1, 128), F32).at[0, 40].set(1.0)

    htab1, astab1, adtab1 = _tc_pre(xp, W1, A_s1, A_d1)
    acc1, den1 = _edge_pass_cached(True, 0, True, True)(
        srcI, dstI, htab1, astab1, adtab1)
    acc1 = acc1.reshape(NC, NP, 128)
    den1 = den1.reshape(NC, NPD, 128)
    stab2, dtab2 = _tc_mid(acc1, den1, b1.reshape(1, 128), W2E, W2D, E8b, P2,
                           c40)
    acc2 = _edge_pass_cached(False, 48, False, False, 128)(
        srcI, dstI, stab2, stab2, dtab2)
    acc2 = acc2.reshape(NC, NP, 128)
    out = _tc_out(acc2, b2.reshape(1, 40))
    return out[:N]
